# baseline (speedup 1.0000x reference)
import functools

import jax
import jax.numpy as jnp
from jax import lax
from jax.experimental import pallas as pl
from jax.experimental.pallas import tpu as pltpu
from jax.experimental.pallas import tpu_sc as plsc

H, D = 8, 16
HID = H * D
IN = 128
OUT = 64
N = 10000
E = 320000
NEG = 0.2

NT = 16
EPT = E // NT
K = 80
NCHUNK = EPT // K
NP = 10112
RPT = NP // NT
BLK = 400


def _proj_body(nt, x_ref, W_ref, b_ref, atts_ref, hs_ref, *al_refs):
    h = jnp.dot(x_ref[...], W_ref[...], preferred_element_type=jnp.float32)
    h = h + b_ref[...]
    ci = lax.broadcasted_iota(jnp.int32, (HID, H), 0)
    hi = lax.broadcasted_iota(jnp.int32, (HID, H), 1)
    G = jnp.where(ci // D == hi, 1.0, 0.0).astype(jnp.float32)
    als = jnp.dot(h * atts_ref[0, :][None, :], G,
                  preferred_element_type=jnp.float32)
    hs_ref[...] = jnp.concatenate([h, als], axis=1)
    for t in range(nt):
        at = atts_ref[t + 1, :][None, :]
        al_refs[t][...] = jnp.dot(h * at, G, preferred_element_type=jnp.float32)


def _proj_call(x, W, b, atts, nt):
    outs = [jax.ShapeDtypeStruct((N, HID + H), jnp.float32)]
    outs += [jax.ShapeDtypeStruct((N, H), jnp.float32)] * nt
    return pl.pallas_call(
        functools.partial(_proj_body, nt),
        grid=(N // BLK,),
        in_specs=[
            pl.BlockSpec((BLK, IN), lambda i: (i, 0)),
            pl.BlockSpec((IN, HID), lambda i: (0, 0)),
            pl.BlockSpec((1, HID), lambda i: (0, 0)),
            pl.BlockSpec((nt + 1, HID), lambda i: (0, 0)),
        ],
        out_specs=[pl.BlockSpec((BLK, HID + H), lambda i: (i, 0))]
        + [pl.BlockSpec((BLK, H), lambda i: (i, 0))] * nt,
        out_shape=outs,
    )(x, W, b, atts)


def _edge_type(src2, dst2, hs_t, ald_t, znum, zden, num_out, den_out,
               sidx, didx, ald_b, hsbuf_b, msg_b,
               sem_i, sem_g, sem_sd, sem_sm, num_sh, den_sh, sid):
    r0 = sid * RPT
    pltpu.sync_copy(znum.at[pl.ds(r0, RPT)], num_sh.at[pl.ds(r0, RPT)])
    pltpu.sync_copy(zden.at[pl.ds(r0, RPT)], den_sh.at[pl.ds(r0, RPT)])
    plsc.subcore_barrier()

    lanes = lax.iota(jnp.int32, 16)
    row_off = lanes // H
    col = lanes - H * row_off

    cbase = sid * NCHUNK

    def issue_idx(ci, b):
        pltpu.async_copy(src2.at[cbase + ci], sidx.at[b], sem_i[b])
        pltpu.async_copy(dst2.at[cbase + ci], didx.at[ci % 4], sem_i[b])

    def wait_idx(b):
        pltpu.make_async_copy(src2.at[cbase], sidx.at[b], sem_i[b]).wait()
        pltpu.make_async_copy(dst2.at[cbase], didx.at[0], sem_i[b]).wait()

    def issue_g(ci, b):
        pltpu.async_copy(hs_t.at[sidx.at[b]], hsbuf_b[b], sem_g[b])
        pltpu.async_copy(ald_t.at[didx.at[ci % 4]], ald_b[b], sem_g[b])

    def wait_g(b):
        pltpu.make_async_copy(hs_t.at[sidx.at[0]], hsbuf_b[b], sem_g[b]).wait()
        pltpu.make_async_copy(ald_t.at[didx.at[0]], ald_b[b], sem_g[b]).wait()

    def issue_s(ci, b):
        dv = didx.at[ci % 4]
        pltpu.async_copy(ald_b[b], den_sh.at[dv], sem_sd[b], add=True)
        pltpu.async_copy(msg_b[b], num_sh.at[dv], sem_sm[b], add=True)

    def wait_sd(b):
        pltpu.make_async_copy(ald_b[b], den_sh.at[didx.at[0]], sem_sd[b]).wait()

    def wait_sm(b):
        pltpu.make_async_copy(msg_b[b], num_sh.at[didx.at[0]], sem_sm[b]).wait()

    def compute(b):
        ald_v = ald_b[b]
        hrows_v, msg_v = hsbuf_b[b], msg_b[b]

        def fused_body(q, carry):
            for pp in range(2):
                p = 2 * q + pp
                r = 2 * p + row_off
                a = (plsc.load_gather(hrows_v, [r, col + HID])
                     + plsc.load_gather(ald_v, [r, col]))
                a = jnp.where(a > 0, a, NEG * a)
                ex2 = jnp.exp(a)
                plsc.store_scatter(ald_v, [r, col], ex2)
                for ee in range(2):
                    e = 2 * p + ee
                    for j in range(H):
                        w = lax.broadcast_in_dim(ex2[ee * H + j], (16,), ())
                        msg_v[e, pl.ds(j * D, D)] = hrows_v[e, pl.ds(j * D, D)] * w
            return carry
        lax.fori_loop(0, K // 4, fused_body, 0)

    def half(c, g, b):
        @pl.when(g > 0)
        def _():
            wait_sm(b)
        if b == 0:
            @pl.when(g > 0)
            def _():
                wait_sd(1 - b)
        else:
            wait_sd(1 - b)
        if b == 0:
            wait_idx(1 - b)
            issue_g(c + 1, 1 - b)
        else:
            @pl.when(g < NCHUNK // 2 - 1)
            def _():
                wait_idx(1 - b)
                issue_g(c + 1, 1 - b)
        wait_g(b)
        @pl.when(g < NCHUNK // 2 - 1)
        def _():
            issue_idx(c + 2, b)
        compute(b)
        issue_s(c, b)

    issue_idx(0, 0)
    issue_idx(1, 1)
    wait_idx(0)
    issue_g(0, 0)

    def pair_body(g, carry):
        half(2 * g, g, 0)
        half(2 * g + 1, g, 1)
        return carry

    lax.fori_loop(0, NCHUNK // 2, pair_body, 0)
    wait_sd(1)
    wait_sm(0)
    wait_sm(1)
    plsc.subcore_barrier()
    pltpu.sync_copy(num_sh.at[pl.ds(r0, RPT)], num_out.at[pl.ds(r0, RPT)])
    pltpu.sync_copy(den_sh.at[pl.ds(r0, RPT)], den_out.at[pl.ds(r0, RPT)])


def _sc_body(src_iu, dst_iu, src_uu, dst_uu, ald_iu, ald_uu,
             hs_item, hs_user, znum, zden, num_iu, den_iu, num_uu, den_uu,
             sidx, didx, ald0, ald1,
             hsbuf0, hsbuf1, msg0, msg1,
             si0, si1, sg0, sg1, sd0, sd1, sm0, sm1, num_sh, den_sh):
    c = lax.axis_index("c")
    sid = lax.axis_index("s")
    ald_b = (ald0, ald1)
    hsbuf_b, msg_b = (hsbuf0, hsbuf1), (msg0, msg1)
    sem_i, sem_g = (si0, si1), (sg0, sg1)
    sem_sd, sem_sm = (sd0, sd1), (sm0, sm1)

    @pl.when(c == 0)
    def _():
        _edge_type(src_iu, dst_iu, hs_item, ald_iu, znum, zden,
                   num_iu, den_iu, sidx, didx, ald_b,
                   hsbuf_b, msg_b, sem_i, sem_g, sem_sd, sem_sm,
                   num_sh, den_sh, sid)

    @pl.when(c == 1)
    def _():
        _edge_type(src_uu, dst_uu, hs_user, ald_uu, znum, zden,
                   num_uu, den_uu, sidx, didx, ald_b,
                   hsbuf_b, msg_b, sem_i, sem_g, sem_sd, sem_sm,
                   num_sh, den_sh, sid)


@functools.cache
def _sc_kernel():
    return pl.kernel(
        _sc_body,
        out_type=[
            jax.ShapeDtypeStruct((NP, HID), jnp.float32),
            jax.ShapeDtypeStruct((NP, H), jnp.float32),
            jax.ShapeDtypeStruct((NP, HID), jnp.float32),
            jax.ShapeDtypeStruct((NP, H), jnp.float32),
        ],
        mesh=plsc.VectorSubcoreMesh(core_axis_name="c", subcore_axis_name="s",
                                    num_cores=2, num_subcores=NT),
        scratch_types=[
            pltpu.VMEM((2, K), jnp.int32),
            pltpu.VMEM((4, K), jnp.int32),
            pltpu.VMEM((K, H), jnp.float32),
            pltpu.VMEM((K, H), jnp.float32),
            pltpu.VMEM((K, HID + H), jnp.float32),
            pltpu.VMEM((K, HID + H), jnp.float32),
            pltpu.VMEM((K, HID), jnp.float32),
            pltpu.VMEM((K, HID), jnp.float32),
            pltpu.SemaphoreType.DMA,
            pltpu.SemaphoreType.DMA,
            pltpu.SemaphoreType.DMA,
            pltpu.SemaphoreType.DMA,
            pltpu.SemaphoreType.DMA,
            pltpu.SemaphoreType.DMA,
            pltpu.SemaphoreType.DMA,
            pltpu.SemaphoreType.DMA,
            pltpu.VMEM_SHARED((NP, HID), jnp.float32),
            pltpu.VMEM_SHARED((NP, H), jnp.float32),
        ],
        compiler_params=pltpu.CompilerParams(use_tc_tiling_on_sc=False,
                                             needs_layout_passes=False),
    )


def _post_body(niu_ref, diu_ref, nuu_ref, duu_ref, Wk_ref, bk_ref, q_ref,
               oiu_ref, ouu_ref, ps_ref):
    hi = lax.broadcasted_iota(jnp.int32, (H, HID), 0)
    ci = lax.broadcasted_iota(jnp.int32, (H, HID), 1)
    G8 = jnp.where(ci // D == hi, 1.0, 0.0).astype(jnp.float32)

    def one(nref, dref, oref):
        den = jnp.dot(dref[...], G8, preferred_element_type=jnp.float32) + 1e-16
        o = jnp.maximum(nref[...] / den, 0.0)
        oref[...] = o
        t = jnp.tanh(jnp.dot(o, Wk_ref[...], preferred_element_type=jnp.float32)
                     + bk_ref[...])
        return jnp.sum(t * q_ref[...])

    s0 = one(niu_ref, diu_ref, oiu_ref)
    s1 = one(nuu_ref, duu_ref, ouu_ref)
    lane = lax.broadcasted_iota(jnp.int32, (1, 1, HID), 2)
    ps_ref[...] = jnp.where(lane == 0, s0, jnp.where(lane == 1, s1, 0.0))


def _post_call(num_iu, den_iu, num_uu, den_uu, W_k, b_k, q):
    return pl.pallas_call(
        _post_body,
        grid=(N // BLK,),
        in_specs=[
            pl.BlockSpec((BLK, HID), lambda i: (i, 0)),
            pl.BlockSpec((BLK, H), lambda i: (i, 0)),
            pl.BlockSpec((BLK, HID), lambda i: (i, 0)),
            pl.BlockSpec((BLK, H), lambda i: (i, 0)),
            pl.BlockSpec((HID, HID), lambda i: (0, 0)),
            pl.BlockSpec((1, HID), lambda i: (0, 0)),
            pl.BlockSpec((1, HID), lambda i: (0, 0)),
        ],
        out_specs=[
            pl.BlockSpec((BLK, HID), lambda i: (i, 0)),
            pl.BlockSpec((BLK, HID), lambda i: (i, 0)),
            pl.BlockSpec((1, 1, HID), lambda i: (i, 0, 0)),
        ],
        out_shape=[
            jax.ShapeDtypeStruct((N, HID), jnp.float32),
            jax.ShapeDtypeStruct((N, HID), jnp.float32),
            jax.ShapeDtypeStruct((N // BLK, 1, HID), jnp.float32),
        ],
    )(num_iu, den_iu, num_uu, den_uu, W_k, b_k, q)


def _final_body(ps_ref, oiu_ref, ouu_ref, Wo_ref, bo_ref, out_ref):
    s = jnp.sum(ps_ref[...], axis=0, keepdims=True) * (1.0 / N)
    lane = lax.broadcasted_iota(jnp.int32, (1, 1, HID), 2)
    s0 = jnp.sum(jnp.where(lane == 0, s, 0.0))
    s1 = jnp.sum(jnp.where(lane == 1, s, 0.0))
    m = jnp.maximum(s0, s1)
    e0 = jnp.exp(s0 - m)
    e1 = jnp.exp(s1 - m)
    a0 = e0 / (e0 + e1)
    a1 = e1 / (e0 + e1)
    comb = a0 * oiu_ref[...] + a1 * ouu_ref[...]
    out_ref[...] = (jnp.dot(comb, Wo_ref[...], preferred_element_type=jnp.float32)
                    + bo_ref[...])


def _final_call(ps, o_iu, o_uu, W_out, b_out):
    return pl.pallas_call(
        _final_body,
        grid=(N // BLK,),
        in_specs=[
            pl.BlockSpec((N // BLK, 1, HID), lambda i: (0, 0, 0)),
            pl.BlockSpec((BLK, HID), lambda i: (i, 0)),
            pl.BlockSpec((BLK, HID), lambda i: (i, 0)),
            pl.BlockSpec((HID, OUT), lambda i: (0, 0)),
            pl.BlockSpec((1, OUT), lambda i: (0, 0)),
        ],
        out_specs=pl.BlockSpec((BLK, OUT), lambda i: (i, 0)),
        out_shape=jax.ShapeDtypeStruct((N, OUT), jnp.float32),
    )(ps, o_iu, o_uu, W_out, b_out)


def kernel(x_user, x_item, edge_index_ui, edge_index_iu, edge_index_uu,
           W_proj_user, b_proj_user, W_proj_item, b_proj_item,
           att_src_ui, att_dst_ui, att_src_iu, att_dst_iu,
           att_src_uu, att_dst_uu, W_k, b_k, q, W_out, b_out):
    atts_user = jnp.stack([att_src_uu.reshape(HID), att_dst_iu.reshape(HID),
                           att_dst_uu.reshape(HID)])
    atts_item = jnp.stack([att_src_iu.reshape(HID), att_src_iu.reshape(HID)])

    hs_user, al_d_iu, al_d_uu = _proj_call(
        x_user, W_proj_user, b_proj_user.reshape(1, HID), atts_user, 2)
    hs_item, _unused_al = _proj_call(
        x_item, W_proj_item, b_proj_item.reshape(1, HID), atts_item, 1)

    znum = jnp.zeros((NP, HID), jnp.float32)
    zden = jnp.zeros((NP, H), jnp.float32)
    num_iu, den_iu, num_uu, den_uu = _sc_kernel()(
        edge_index_iu[0].reshape(E // K, K), edge_index_iu[1].reshape(E // K, K),
        edge_index_uu[0].reshape(E // K, K), edge_index_uu[1].reshape(E // K, K),
        al_d_iu, al_d_uu, hs_item, hs_user, znum, zden)
    num_iu, den_iu = num_iu[:N], den_iu[:N]
    num_uu, den_uu = num_uu[:N], den_uu[:N]

    o_iu, o_uu, ps = _post_call(num_iu, den_iu, num_uu, den_uu,
                                W_k, b_k.reshape(1, HID), q.reshape(1, HID))
    return _final_call(ps, o_iu, o_uu, W_out, b_out.reshape(1, OUT))


# speedup vs baseline: 1.0410x; 1.0410x over previous
import functools

import jax
import jax.numpy as jnp
from jax import lax
from jax.experimental import pallas as pl
from jax.experimental.pallas import tpu as pltpu
from jax.experimental.pallas import tpu_sc as plsc

H, D = 8, 16
HID = H * D
IN = 128
OUT = 64
N = 10000
E = 320000
NEG = 0.2

NT = 16
EPT = E // NT
K = 80
NCHUNK = EPT // K
NP = 10112
RPT = NP // NT
BLK = 400
BLKP = 632


def _proj_body(nt, x_ref, W_ref, b_ref, atts_ref, h_ref, *al_refs):
    h = jnp.dot(x_ref[...], W_ref[...], preferred_element_type=jnp.float32)
    h = h + b_ref[...]
    h_ref[...] = h
    ci = lax.broadcasted_iota(jnp.int32, (HID, H), 0)
    hi = lax.broadcasted_iota(jnp.int32, (HID, H), 1)
    G = jnp.where(ci // D == hi, 1.0, 0.0).astype(jnp.float32)
    for t in range(nt):
        at = atts_ref[t, :][None, :]
        al_refs[t][...] = jnp.dot(h * at, G, preferred_element_type=jnp.float32)


def _proj_call(x, W, b, atts, nt):
    outs = [jax.ShapeDtypeStruct((N, HID), jnp.float32)]
    outs += [jax.ShapeDtypeStruct((N, H), jnp.float32)] * nt
    return pl.pallas_call(
        functools.partial(_proj_body, nt),
        grid=(N // BLK,),
        in_specs=[
            pl.BlockSpec((BLK, IN), lambda i: (i, 0)),
            pl.BlockSpec((IN, HID), lambda i: (0, 0)),
            pl.BlockSpec((1, HID), lambda i: (0, 0)),
            pl.BlockSpec((nt, HID), lambda i: (0, 0)),
        ],
        out_specs=[pl.BlockSpec((BLK, HID), lambda i: (i, 0))]
        + [pl.BlockSpec((BLK, H), lambda i: (i, 0))] * nt,
        out_shape=outs,
    )(x, W, b, atts)


def _edge_type(src2, dst2, als_t, ald_t, h_t, znum, zden, num_out, den_out,
               sidx, didx, als_b, ald_b, hbuf_b, msg_b,
               sem_i, sem_g, sem_sd, sem_sm, num_sh, den_sh, sid):
    r0 = sid * RPT
    pltpu.sync_copy(znum.at[pl.ds(r0, RPT)], num_sh.at[pl.ds(r0, RPT)])
    pltpu.sync_copy(zden.at[pl.ds(r0, RPT)], den_sh.at[pl.ds(r0, RPT)])
    plsc.subcore_barrier()

    lanes = lax.iota(jnp.int32, 16)
    row_off = lanes // H
    col = lanes - H * row_off

    cbase = sid * NCHUNK

    def issue_idx(ci, b):
        pltpu.async_copy(src2.at[cbase + ci], sidx.at[b], sem_i[b])
        pltpu.async_copy(dst2.at[cbase + ci], didx.at[ci % 4], sem_i[b])

    def wait_idx(b):
        pltpu.make_async_copy(src2.at[cbase], sidx.at[b], sem_i[b]).wait()
        pltpu.make_async_copy(dst2.at[cbase], didx.at[0], sem_i[b]).wait()

    def issue_g(ci, b):
        sv = sidx.at[b]
        pltpu.async_copy(als_t.at[sv], als_b[b], sem_g[b])
        pltpu.async_copy(ald_t.at[didx.at[ci % 4]], ald_b[b], sem_g[b])
        pltpu.async_copy(h_t.at[sv], hbuf_b[b], sem_g[b])

    def wait_g(b):
        z = sidx.at[0]
        pltpu.make_async_copy(als_t.at[z], als_b[b], sem_g[b]).wait()
        pltpu.make_async_copy(ald_t.at[z], ald_b[b], sem_g[b]).wait()
        pltpu.make_async_copy(h_t.at[z], hbuf_b[b], sem_g[b]).wait()

    def issue_s(ci, b):
        dv = didx.at[ci % 4]
        pltpu.async_copy(ald_b[b], den_sh.at[dv], sem_sd[b], add=True)
        pltpu.async_copy(msg_b[b], num_sh.at[dv], sem_sm[b], add=True)

    def wait_sd(b):
        pltpu.make_async_copy(ald_b[b], den_sh.at[didx.at[0]], sem_sd[b]).wait()

    def wait_sm(b):
        pltpu.make_async_copy(msg_b[b], num_sh.at[didx.at[0]], sem_sm[b]).wait()

    def compute(b):
        als_v, ald_v = als_b[b], ald_b[b]
        hrows_v, msg_v = hbuf_b[b], msg_b[b]

        def fused_body(q, carry):
            for pp in range(2):
                p = 2 * q + pp
                r = 2 * p + row_off
                a = plsc.load_gather(als_v, [r, col]) + plsc.load_gather(ald_v, [r, col])
                a = jnp.where(a > 0, a, NEG * a)
                ex2 = jnp.exp(a)
                plsc.store_scatter(ald_v, [r, col], ex2)
                for ee in range(2):
                    e = 2 * p + ee
                    for j in range(H):
                        w = lax.broadcast_in_dim(ex2[ee * H + j], (16,), ())
                        msg_v[e, pl.ds(j * D, D)] = hrows_v[e, pl.ds(j * D, D)] * w
            return carry
        lax.fori_loop(0, K // 4, fused_body, 0)

    def half(c, g, b):
        @pl.when(g > 0)
        def _():
            wait_sm(b)
        if b == 0:
            @pl.when(g > 0)
            def _():
                wait_sd(1 - b)
        else:
            wait_sd(1 - b)
        if b == 0:
            wait_idx(1 - b)
            issue_g(c + 1, 1 - b)
        else:
            @pl.when(g < NCHUNK // 2 - 1)
            def _():
                wait_idx(1 - b)
                issue_g(c + 1, 1 - b)
        wait_g(b)
        @pl.when(g < NCHUNK // 2 - 1)
        def _():
            issue_idx(c + 2, b)
        compute(b)
        issue_s(c, b)

    issue_idx(0, 0)
    issue_idx(1, 1)
    wait_idx(0)
    issue_g(0, 0)

    def pair_body(g, carry):
        half(2 * g, g, 0)
        half(2 * g + 1, g, 1)
        return carry

    lax.fori_loop(0, NCHUNK // 2, pair_body, 0)
    wait_sd(1)
    wait_sm(0)
    wait_sm(1)
    plsc.subcore_barrier()
    pltpu.sync_copy(num_sh.at[pl.ds(r0, RPT)], num_out.at[pl.ds(r0, RPT)])
    pltpu.sync_copy(den_sh.at[pl.ds(r0, RPT)], den_out.at[pl.ds(r0, RPT)])


def _sc_body(src_iu, dst_iu, src_uu, dst_uu, als_iu, ald_iu, als_uu, ald_uu,
             h_item, h_user, znum, zden, num_iu, den_iu, num_uu, den_uu,
             sidx, didx, als0, als1, ald0, ald1,
             hbuf0, hbuf1, msg0, msg1,
             si0, si1, sg0, sg1, sd0, sd1, sm0, sm1, num_sh, den_sh):
    c = lax.axis_index("c")
    sid = lax.axis_index("s")
    als_b, ald_b = (als0, als1), (ald0, ald1)
    hbuf_b, msg_b = (hbuf0, hbuf1), (msg0, msg1)
    sem_i, sem_g = (si0, si1), (sg0, sg1)
    sem_sd, sem_sm = (sd0, sd1), (sm0, sm1)

    @pl.when(c == 0)
    def _():
        _edge_type(src_iu, dst_iu, als_iu, ald_iu, h_item, znum, zden,
                   num_iu, den_iu, sidx, didx, als_b, ald_b,
                   hbuf_b, msg_b, sem_i, sem_g, sem_sd, sem_sm,
                   num_sh, den_sh, sid)

    @pl.when(c == 1)
    def _():
        _edge_type(src_uu, dst_uu, als_uu, ald_uu, h_user, znum, zden,
                   num_uu, den_uu, sidx, didx, als_b, ald_b,
                   hbuf_b, msg_b, sem_i, sem_g, sem_sd, sem_sm,
                   num_sh, den_sh, sid)


@functools.cache
def _sc_kernel():
    return pl.kernel(
        _sc_body,
        out_type=[
            jax.ShapeDtypeStruct((NP, HID), jnp.float32),
            jax.ShapeDtypeStruct((NP, H), jnp.float32),
            jax.ShapeDtypeStruct((NP, HID), jnp.float32),
            jax.ShapeDtypeStruct((NP, H), jnp.float32),
        ],
        mesh=plsc.VectorSubcoreMesh(core_axis_name="c", subcore_axis_name="s",
                                    num_cores=2, num_subcores=NT),
        scratch_types=[
            pltpu.VMEM((2, K), jnp.int32),
            pltpu.VMEM((4, K), jnp.int32),
            pltpu.VMEM((K, H), jnp.float32),
            pltpu.VMEM((K, H), jnp.float32),
            pltpu.VMEM((K, H), jnp.float32),
            pltpu.VMEM((K, H), jnp.float32),
            pltpu.VMEM((K, HID), jnp.float32),
            pltpu.VMEM((K, HID), jnp.float32),
            pltpu.VMEM((K, HID), jnp.float32),
            pltpu.VMEM((K, HID), jnp.float32),
            pltpu.SemaphoreType.DMA,
            pltpu.SemaphoreType.DMA,
            pltpu.SemaphoreType.DMA,
            pltpu.SemaphoreType.DMA,
            pltpu.SemaphoreType.DMA,
            pltpu.SemaphoreType.DMA,
            pltpu.SemaphoreType.DMA,
            pltpu.SemaphoreType.DMA,
            pltpu.VMEM_SHARED((NP, HID), jnp.float32),
            pltpu.VMEM_SHARED((NP, H), jnp.float32),
        ],
        compiler_params=pltpu.CompilerParams(use_tc_tiling_on_sc=False,
                                             needs_layout_passes=False),
    )


def _post_body(niu_ref, diu_ref, nuu_ref, duu_ref, Wk_ref, bk_ref, q_ref,
               oiu_ref, ouu_ref, ps_ref):
    hi = lax.broadcasted_iota(jnp.int32, (H, HID), 0)
    ci = lax.broadcasted_iota(jnp.int32, (H, HID), 1)
    G8 = jnp.where(ci // D == hi, 1.0, 0.0).astype(jnp.float32)
    i = pl.program_id(0)
    rows = i * BLKP + lax.broadcasted_iota(jnp.int32, (BLKP, HID), 0)
    valid = rows < N

    def one(nref, dref, oref):
        den = jnp.dot(dref[...], G8, preferred_element_type=jnp.float32) + 1e-16
        o = jnp.maximum(nref[...] / den, 0.0)
        oref[...] = o
        t = jnp.tanh(jnp.dot(o, Wk_ref[...], preferred_element_type=jnp.float32)
                     + bk_ref[...])
        return jnp.sum(jnp.where(valid, t * q_ref[...], 0.0))

    s0 = one(niu_ref, diu_ref, oiu_ref)
    s1 = one(nuu_ref, duu_ref, ouu_ref)
    lane = lax.broadcasted_iota(jnp.int32, (1, 1, HID), 2)
    ps_ref[...] = jnp.where(lane == 0, s0, jnp.where(lane == 1, s1, 0.0))


def _post_call(num_iu, den_iu, num_uu, den_uu, W_k, b_k, q):
    return pl.pallas_call(
        _post_body,
        grid=(NP // BLKP,),
        in_specs=[
            pl.BlockSpec((BLKP, HID), lambda i: (i, 0)),
            pl.BlockSpec((BLKP, H), lambda i: (i, 0)),
            pl.BlockSpec((BLKP, HID), lambda i: (i, 0)),
            pl.BlockSpec((BLKP, H), lambda i: (i, 0)),
            pl.BlockSpec((HID, HID), lambda i: (0, 0)),
            pl.BlockSpec((1, HID), lambda i: (0, 0)),
            pl.BlockSpec((1, HID), lambda i: (0, 0)),
        ],
        out_specs=[
            pl.BlockSpec((BLKP, HID), lambda i: (i, 0)),
            pl.BlockSpec((BLKP, HID), lambda i: (i, 0)),
            pl.BlockSpec((1, 1, HID), lambda i: (i, 0, 0)),
        ],
        out_shape=[
            jax.ShapeDtypeStruct((NP, HID), jnp.float32),
            jax.ShapeDtypeStruct((NP, HID), jnp.float32),
            jax.ShapeDtypeStruct((NP // BLKP, 1, HID), jnp.float32),
        ],
    )(num_iu, den_iu, num_uu, den_uu, W_k, b_k, q)


def _final_body(ps_ref, oiu_ref, ouu_ref, Wo_ref, bo_ref, out_ref):
    s = jnp.sum(ps_ref[...], axis=0, keepdims=True) * (1.0 / N)
    lane = lax.broadcasted_iota(jnp.int32, (1, 1, HID), 2)
    s0 = jnp.sum(jnp.where(lane == 0, s, 0.0))
    s1 = jnp.sum(jnp.where(lane == 1, s, 0.0))
    m = jnp.maximum(s0, s1)
    e0 = jnp.exp(s0 - m)
    e1 = jnp.exp(s1 - m)
    a0 = e0 / (e0 + e1)
    a1 = e1 / (e0 + e1)
    comb = a0 * oiu_ref[...] + a1 * ouu_ref[...]
    out_ref[...] = (jnp.dot(comb, Wo_ref[...], preferred_element_type=jnp.float32)
                    + bo_ref[...])


def _final_call(ps, o_iu, o_uu, W_out, b_out):
    return pl.pallas_call(
        _final_body,
        grid=(NP // BLKP,),
        in_specs=[
            pl.BlockSpec((NP // BLKP, 1, HID), lambda i: (0, 0, 0)),
            pl.BlockSpec((BLKP, HID), lambda i: (i, 0)),
            pl.BlockSpec((BLKP, HID), lambda i: (i, 0)),
            pl.BlockSpec((HID, OUT), lambda i: (0, 0)),
            pl.BlockSpec((1, OUT), lambda i: (0, 0)),
        ],
        out_specs=pl.BlockSpec((BLKP, OUT), lambda i: (i, 0)),
        out_shape=jax.ShapeDtypeStruct((NP, OUT), jnp.float32),
    )(ps, o_iu, o_uu, W_out, b_out)


def kernel(x_user, x_item, edge_index_ui, edge_index_iu, edge_index_uu,
           W_proj_user, b_proj_user, W_proj_item, b_proj_item,
           att_src_ui, att_dst_ui, att_src_iu, att_dst_iu,
           att_src_uu, att_dst_uu, W_k, b_k, q, W_out, b_out):
    atts_user = jnp.stack([att_dst_iu.reshape(HID), att_src_uu.reshape(HID),
                           att_dst_uu.reshape(HID)])
    atts_item = att_src_iu.reshape(1, HID)

    h_user, al_d_iu, al_s_uu, al_d_uu = _proj_call(
        x_user, W_proj_user, b_proj_user.reshape(1, HID), atts_user, 3)
    h_item, al_s_iu = _proj_call(
        x_item, W_proj_item, b_proj_item.reshape(1, HID), atts_item, 1)

    znum = jnp.zeros((NP, HID), jnp.float32)
    zden = jnp.zeros((NP, H), jnp.float32)
    num_iu, den_iu, num_uu, den_uu = _sc_kernel()(
        edge_index_iu[0].reshape(E // K, K), edge_index_iu[1].reshape(E // K, K),
        edge_index_uu[0].reshape(E // K, K), edge_index_uu[1].reshape(E // K, K),
        al_s_iu, al_d_iu, al_s_uu, al_d_uu, h_item, h_user, znum, zden)

    o_iu, o_uu, ps = _post_call(num_iu, den_iu, num_uu, den_uu,
                                W_k, b_k.reshape(1, HID), q.reshape(1, HID))
    return _final_call(ps, o_iu, o_uu, W_out, b_out.reshape(1, OUT))[:N]


# speedup vs baseline: 1.0458x; 1.0046x over previous
import functools

import jax
import jax.numpy as jnp
from jax import lax
from jax.experimental import pallas as pl
from jax.experimental.pallas import tpu as pltpu
from jax.experimental.pallas import tpu_sc as plsc

H, D = 8, 16
HID = H * D
IN = 128
OUT = 64
N = 10000
E = 320000
NEG = 0.2

NT = 16
EPT = E // NT
K = 80
NCHUNK = EPT // K
NP = 10112
RPT = NP // NT
BLK = 400
BLKP = 632


def _proj_body(nt, x_ref, W_ref, b_ref, atts_ref, h_ref, *al_refs):
    h = jnp.dot(x_ref[...], W_ref[...], preferred_element_type=jnp.float32)
    h = h + b_ref[...]
    h_ref[...] = h
    ci = lax.broadcasted_iota(jnp.int32, (HID, H), 0)
    hi = lax.broadcasted_iota(jnp.int32, (HID, H), 1)
    G = jnp.where(ci // D == hi, 1.0, 0.0).astype(jnp.float32)
    for t in range(nt):
        at = atts_ref[t, :][None, :]
        al_refs[t][...] = jnp.dot(h * at, G, preferred_element_type=jnp.float32)


def _proj_call(x, W, b, atts, nt):
    outs = [jax.ShapeDtypeStruct((N, HID), jnp.float32)]
    outs += [jax.ShapeDtypeStruct((N, H), jnp.float32)] * nt
    return pl.pallas_call(
        functools.partial(_proj_body, nt),
        grid=(N // BLK,),
        in_specs=[
            pl.BlockSpec((BLK, IN), lambda i: (i, 0)),
            pl.BlockSpec((IN, HID), lambda i: (0, 0)),
            pl.BlockSpec((1, HID), lambda i: (0, 0)),
            pl.BlockSpec((nt, HID), lambda i: (0, 0)),
        ],
        out_specs=[pl.BlockSpec((BLK, HID), lambda i: (i, 0))]
        + [pl.BlockSpec((BLK, H), lambda i: (i, 0))] * nt,
        out_shape=outs,
    )(x, W, b, atts)


def _edge_type(src2, dst2, als_t, ald_t, h_t, znum, zden, num_out, den_out,
               sidx, didx, als_b, ald_b, hbuf_b, msg_b,
               sem_i, sem_g, sem_sd, sem_sm, num_sh, den_sh, sid):
    r0 = sid * RPT
    pltpu.sync_copy(znum.at[pl.ds(r0, RPT)], num_sh.at[pl.ds(r0, RPT)])
    pltpu.sync_copy(zden.at[pl.ds(r0, RPT)], den_sh.at[pl.ds(r0, RPT)])
    plsc.subcore_barrier()

    lanes = lax.iota(jnp.int32, 16)
    row_off = lanes // H
    col = lanes - H * row_off

    cbase = sid * NCHUNK

    def issue_idx(ci, b):
        pltpu.async_copy(src2.at[cbase + ci], sidx.at[b], sem_i[b])
        pltpu.async_copy(dst2.at[cbase + ci], didx.at[ci % 4], sem_i[b])

    def wait_idx(b):
        pltpu.make_async_copy(src2.at[cbase], sidx.at[b], sem_i[b]).wait()
        pltpu.make_async_copy(dst2.at[cbase], didx.at[0], sem_i[b]).wait()

    def issue_g(ci, b):
        sv = sidx.at[b]
        pltpu.async_copy(als_t.at[sv], als_b[b], sem_g[b])
        pltpu.async_copy(ald_t.at[didx.at[ci % 4]], ald_b[b], sem_g[b])
        pltpu.async_copy(h_t.at[sv], hbuf_b[b], sem_g[b])

    def wait_g(b):
        z = sidx.at[0]
        pltpu.make_async_copy(als_t.at[z], als_b[b], sem_g[b]).wait()
        pltpu.make_async_copy(ald_t.at[z], ald_b[b], sem_g[b]).wait()
        pltpu.make_async_copy(h_t.at[z], hbuf_b[b], sem_g[b]).wait()

    def issue_s(ci, b):
        dv = didx.at[ci % 4]
        pltpu.async_copy(ald_b[b], den_sh.at[dv], sem_sd[b], add=True)
        pltpu.async_copy(msg_b[b], num_sh.at[dv], sem_sm[b], add=True)

    def wait_sd(b):
        pltpu.make_async_copy(ald_b[b], den_sh.at[didx.at[0]], sem_sd[b]).wait()

    def wait_sm(b):
        pltpu.make_async_copy(msg_b[b], num_sh.at[didx.at[0]], sem_sm[b]).wait()

    def compute(b):
        als_v, ald_v = als_b[b], ald_b[b]
        hrows_v, msg_v = hbuf_b[b], msg_b[b]

        def fused_body(q, carry):
            for pp in range(4):
                p = 4 * q + pp
                r = 2 * p + row_off
                a = plsc.load_gather(als_v, [r, col]) + plsc.load_gather(ald_v, [r, col])
                a = jnp.where(a > 0, a, NEG * a)
                ex2 = jnp.exp(a)
                plsc.store_scatter(ald_v, [r, col], ex2)
                for ee in range(2):
                    e = 2 * p + ee
                    for j in range(H):
                        w = lax.broadcast_in_dim(ex2[ee * H + j], (16,), ())
                        msg_v[e, pl.ds(j * D, D)] = hrows_v[e, pl.ds(j * D, D)] * w
            return carry
        lax.fori_loop(0, K // 8, fused_body, 0)

    def half(c, g, b):
        @pl.when(g > 0)
        def _():
            wait_sm(b)
        if b == 0:
            @pl.when(g > 0)
            def _():
                wait_sd(1 - b)
        else:
            wait_sd(1 - b)
        if b == 0:
            wait_idx(1 - b)
            issue_g(c + 1, 1 - b)
        else:
            @pl.when(g < NCHUNK // 2 - 1)
            def _():
                wait_idx(1 - b)
                issue_g(c + 1, 1 - b)
        wait_g(b)
        @pl.when(g < NCHUNK // 2 - 1)
        def _():
            issue_idx(c + 2, b)
        compute(b)
        issue_s(c, b)

    issue_idx(0, 0)
    issue_idx(1, 1)
    wait_idx(0)
    issue_g(0, 0)

    def pair_body(g, carry):
        half(2 * g, g, 0)
        half(2 * g + 1, g, 1)
        return carry

    lax.fori_loop(0, NCHUNK // 2, pair_body, 0)
    wait_sd(1)
    wait_sm(0)
    wait_sm(1)
    plsc.subcore_barrier()
    pltpu.sync_copy(num_sh.at[pl.ds(r0, RPT)], num_out.at[pl.ds(r0, RPT)])
    pltpu.sync_copy(den_sh.at[pl.ds(r0, RPT)], den_out.at[pl.ds(r0, RPT)])


def _sc_body(src_iu, dst_iu, src_uu, dst_uu, als_iu, ald_iu, als_uu, ald_uu,
             h_item, h_user, znum, zden, num_iu, den_iu, num_uu, den_uu,
             sidx, didx, als0, als1, ald0, ald1,
             hbuf0, hbuf1, msg0, msg1,
             si0, si1, sg0, sg1, sd0, sd1, sm0, sm1, num_sh, den_sh):
    c = lax.axis_index("c")
    sid = lax.axis_index("s")
    als_b, ald_b = (als0, als1), (ald0, ald1)
    hbuf_b, msg_b = (hbuf0, hbuf1), (msg0, msg1)
    sem_i, sem_g = (si0, si1), (sg0, sg1)
    sem_sd, sem_sm = (sd0, sd1), (sm0, sm1)

    @pl.when(c == 0)
    def _():
        _edge_type(src_iu, dst_iu, als_iu, ald_iu, h_item, znum, zden,
                   num_iu, den_iu, sidx, didx, als_b, ald_b,
                   hbuf_b, msg_b, sem_i, sem_g, sem_sd, sem_sm,
                   num_sh, den_sh, sid)

    @pl.when(c == 1)
    def _():
        _edge_type(src_uu, dst_uu, als_uu, ald_uu, h_user, znum, zden,
                   num_uu, den_uu, sidx, didx, als_b, ald_b,
                   hbuf_b, msg_b, sem_i, sem_g, sem_sd, sem_sm,
                   num_sh, den_sh, sid)


@functools.cache
def _sc_kernel():
    return pl.kernel(
        _sc_body,
        out_type=[
            jax.ShapeDtypeStruct((NP, HID), jnp.float32),
            jax.ShapeDtypeStruct((NP, H), jnp.float32),
            jax.ShapeDtypeStruct((NP, HID), jnp.float32),
            jax.ShapeDtypeStruct((NP, H), jnp.float32),
        ],
        mesh=plsc.VectorSubcoreMesh(core_axis_name="c", subcore_axis_name="s",
                                    num_cores=2, num_subcores=NT),
        scratch_types=[
            pltpu.VMEM((2, K), jnp.int32),
            pltpu.VMEM((4, K), jnp.int32),
            pltpu.VMEM((K, H), jnp.float32),
            pltpu.VMEM((K, H), jnp.float32),
            pltpu.VMEM((K, H), jnp.float32),
            pltpu.VMEM((K, H), jnp.float32),
            pltpu.VMEM((K, HID), jnp.float32),
            pltpu.VMEM((K, HID), jnp.float32),
            pltpu.VMEM((K, HID), jnp.float32),
            pltpu.VMEM((K, HID), jnp.float32),
            pltpu.SemaphoreType.DMA,
            pltpu.SemaphoreType.DMA,
            pltpu.SemaphoreType.DMA,
            pltpu.SemaphoreType.DMA,
            pltpu.SemaphoreType.DMA,
            pltpu.SemaphoreType.DMA,
            pltpu.SemaphoreType.DMA,
            pltpu.SemaphoreType.DMA,
            pltpu.VMEM_SHARED((NP, HID), jnp.float32),
            pltpu.VMEM_SHARED((NP, H), jnp.float32),
        ],
        compiler_params=pltpu.CompilerParams(use_tc_tiling_on_sc=False,
                                             needs_layout_passes=False),
    )


def _post_body(niu_ref, diu_ref, nuu_ref, duu_ref, Wk_ref, bk_ref, q_ref,
               oiu_ref, ouu_ref, ps_ref):
    hi = lax.broadcasted_iota(jnp.int32, (H, HID), 0)
    ci = lax.broadcasted_iota(jnp.int32, (H, HID), 1)
    G8 = jnp.where(ci // D == hi, 1.0, 0.0).astype(jnp.float32)
    i = pl.program_id(0)
    rows = i * BLKP + lax.broadcasted_iota(jnp.int32, (BLKP, HID), 0)
    valid = rows < N

    def one(nref, dref, oref):
        den = jnp.dot(dref[...], G8, preferred_element_type=jnp.float32) + 1e-16
        o = jnp.maximum(nref[...] / den, 0.0)
        oref[...] = o
        t = jnp.tanh(jnp.dot(o, Wk_ref[...], preferred_element_type=jnp.float32)
                     + bk_ref[...])
        return jnp.sum(jnp.where(valid, t * q_ref[...], 0.0))

    s0 = one(niu_ref, diu_ref, oiu_ref)
    s1 = one(nuu_ref, duu_ref, ouu_ref)
    lane = lax.broadcasted_iota(jnp.int32, (1, 1, HID), 2)
    ps_ref[...] = jnp.where(lane == 0, s0, jnp.where(lane == 1, s1, 0.0))


def _post_call(num_iu, den_iu, num_uu, den_uu, W_k, b_k, q):
    return pl.pallas_call(
        _post_body,
        grid=(NP // BLKP,),
        in_specs=[
            pl.BlockSpec((BLKP, HID), lambda i: (i, 0)),
            pl.BlockSpec((BLKP, H), lambda i: (i, 0)),
            pl.BlockSpec((BLKP, HID), lambda i: (i, 0)),
            pl.BlockSpec((BLKP, H), lambda i: (i, 0)),
            pl.BlockSpec((HID, HID), lambda i: (0, 0)),
            pl.BlockSpec((1, HID), lambda i: (0, 0)),
            pl.BlockSpec((1, HID), lambda i: (0, 0)),
        ],
        out_specs=[
            pl.BlockSpec((BLKP, HID), lambda i: (i, 0)),
            pl.BlockSpec((BLKP, HID), lambda i: (i, 0)),
            pl.BlockSpec((1, 1, HID), lambda i: (i, 0, 0)),
        ],
        out_shape=[
            jax.ShapeDtypeStruct((NP, HID), jnp.float32),
            jax.ShapeDtypeStruct((NP, HID), jnp.float32),
            jax.ShapeDtypeStruct((NP // BLKP, 1, HID), jnp.float32),
        ],
    )(num_iu, den_iu, num_uu, den_uu, W_k, b_k, q)


def _final_body(ps_ref, oiu_ref, ouu_ref, Wo_ref, bo_ref, out_ref):
    s = jnp.sum(ps_ref[...], axis=0, keepdims=True) * (1.0 / N)
    lane = lax.broadcasted_iota(jnp.int32, (1, 1, HID), 2)
    s0 = jnp.sum(jnp.where(lane == 0, s, 0.0))
    s1 = jnp.sum(jnp.where(lane == 1, s, 0.0))
    m = jnp.maximum(s0, s1)
    e0 = jnp.exp(s0 - m)
    e1 = jnp.exp(s1 - m)
    a0 = e0 / (e0 + e1)
    a1 = e1 / (e0 + e1)
    comb = a0 * oiu_ref[...] + a1 * ouu_ref[...]
    out_ref[...] = (jnp.dot(comb, Wo_ref[...], preferred_element_type=jnp.float32)
                    + bo_ref[...])


def _final_call(ps, o_iu, o_uu, W_out, b_out):
    return pl.pallas_call(
        _final_body,
        grid=(NP // BLKP,),
        in_specs=[
            pl.BlockSpec((NP // BLKP, 1, HID), lambda i: (0, 0, 0)),
            pl.BlockSpec((BLKP, HID), lambda i: (i, 0)),
            pl.BlockSpec((BLKP, HID), lambda i: (i, 0)),
            pl.BlockSpec((HID, OUT), lambda i: (0, 0)),
            pl.BlockSpec((1, OUT), lambda i: (0, 0)),
        ],
        out_specs=pl.BlockSpec((BLKP, OUT), lambda i: (i, 0)),
        out_shape=jax.ShapeDtypeStruct((NP, OUT), jnp.float32),
    )(ps, o_iu, o_uu, W_out, b_out)


def kernel(x_user, x_item, edge_index_ui, edge_index_iu, edge_index_uu,
           W_proj_user, b_proj_user, W_proj_item, b_proj_item,
           att_src_ui, att_dst_ui, att_src_iu, att_dst_iu,
           att_src_uu, att_dst_uu, W_k, b_k, q, W_out, b_out):
    atts_user = jnp.stack([att_dst_iu.reshape(HID), att_src_uu.reshape(HID),
                           att_dst_uu.reshape(HID)])
    atts_item = att_src_iu.reshape(1, HID)

    h_user, al_d_iu, al_s_uu, al_d_uu = _proj_call(
        x_user, W_proj_user, b_proj_user.reshape(1, HID), atts_user, 3)
    h_item, al_s_iu = _proj_call(
        x_item, W_proj_item, b_proj_item.reshape(1, HID), atts_item, 1)

    znum = jnp.zeros((NP, HID), jnp.float32)
    zden = jnp.zeros((NP, H), jnp.float32)
    num_iu, den_iu, num_uu, den_uu = _sc_kernel()(
        edge_index_iu[0].reshape(E // K, K), edge_index_iu[1].reshape(E // K, K),
        edge_index_uu[0].reshape(E // K, K), edge_index_uu[1].reshape(E // K, K),
        al_s_iu, al_d_iu, al_s_uu, al_d_uu, h_item, h_user, znum, zden)

    o_iu, o_uu, ps = _post_call(num_iu, den_iu, num_uu, den_uu,
                                W_k, b_k.reshape(1, HID), q.reshape(1, HID))
    return _final_call(ps, o_iu, o_uu, W_out, b_out.reshape(1, OUT))[:N]
